# baseline (speedup 1.0000x reference)
import functools

import jax
import jax.numpy as jnp
from jax import lax
from jax.experimental import pallas as pl
from jax.experimental.pallas import tpu as pltpu
from jax.experimental.pallas import tpu_sc as plsc

N = 10000
E = 320000
D = 128
R = 16
NB = 8

NC = 2
NS = 16
NW = NC * NS
EPT = E // NW
C = 80
K = EPT // C
NPAD = 10240
RPS = NPAD // NS


def _basis_body(wc_ref, wf_ref, o_ref):
    o_ref[...] = jnp.dot(wc_ref[...], wf_ref[...],
                         preferred_element_type=jnp.float32)


def _basis(w_comp, weight_flat):
    return pl.pallas_call(
        _basis_body,
        out_shape=jax.ShapeDtypeStruct((R, D * D), jnp.float32),
    )(w_comp, weight_flat)


TN = 400


def _transform_body(h_ref, w_ref, rw_ref, bias_ref, t_ref, root_ref):
    r = pl.program_id(1)
    h = h_ref[...].astype(jnp.bfloat16)
    t_ref[0] = jnp.dot(h, w_ref[0].astype(jnp.bfloat16),
                       preferred_element_type=jnp.float32)

    @pl.when(r == 0)
    def _():
        root_ref[...] = (jnp.dot(h, rw_ref[...].astype(jnp.bfloat16),
                                 preferred_element_type=jnp.float32)
                         + bias_ref[...])


def _transform(h, w3, root_weight, bias2d):
    return pl.pallas_call(
        _transform_body,
        grid=(N // TN, R),
        in_specs=[
            pl.BlockSpec((TN, D), lambda i, r: (i, 0)),
            pl.BlockSpec((1, D, D), lambda i, r: (r, 0, 0)),
            pl.BlockSpec((D, D), lambda i, r: (0, 0)),
            pl.BlockSpec((1, D), lambda i, r: (0, 0)),
        ],
        out_specs=[
            pl.BlockSpec((1, TN, D), lambda i, r: (r, i, 0)),
            pl.BlockSpec((TN, D), lambda i, r: (i, 0)),
        ],
        out_shape=[
            jax.ShapeDtypeStruct((R, N, D), jnp.float32),
            jax.ShapeDtypeStruct((N, D), jnp.float32),
        ],
    )(h, w3, root_weight, bias2d)


def _sc_body(table_hbm, key2_hbm, ei4_hbm, zrows_hbm, out_hbm,
             keys1, dst2, rows0, rows1, acc, semg0, semg1):
    c = lax.axis_index("c")
    s = lax.axis_index("s")
    wid = s * NC + c
    pltpu.sync_copy(key2_hbm.at[wid], keys1)
    pltpu.sync_copy(ei4_hbm.at[1, wid], dst2)
    pltpu.sync_copy(zrows_hbm, acc.at[pl.ds(s * RPS, RPS)])
    plsc.subcore_barrier()

    pltpu.async_copy(table_hbm.at[keys1.at[pl.ds(0, C)]], rows0, semg0)

    def pair(p, carry):
        j0 = 2 * p
        pltpu.async_copy(table_hbm.at[keys1.at[pl.ds((j0 + 1) * C, C)]], rows1, semg1)
        pltpu.make_async_copy(table_hbm.at[keys1.at[pl.ds(j0 * C, C)]], rows0, semg0).wait()
        pltpu.sync_copy(rows0, acc.at[dst2.at[j0]], add=True)
        pltpu.async_copy(table_hbm.at[keys1.at[pl.ds((j0 + 2) * C, C)]], rows0, semg0)
        pltpu.make_async_copy(table_hbm.at[keys1.at[pl.ds((j0 + 1) * C, C)]], rows1,
                              semg1).wait()
        pltpu.sync_copy(rows1, acc.at[dst2.at[j0 + 1]], add=True)
        return carry

    lax.fori_loop(0, (K - 1) // 2, pair, 0)
    pltpu.make_async_copy(table_hbm.at[keys1.at[pl.ds((K - 1) * C, C)]], rows0, semg0).wait()
    pltpu.sync_copy(rows0, acc.at[dst2.at[K - 1]], add=True)
    plsc.subcore_barrier()
    pltpu.sync_copy(acc.at[pl.ds(s * RPS, RPS)],
                    out_hbm.at[c, pl.ds(s * RPS, RPS)])


@functools.cache
def _sc_edge_agg():
    mesh = plsc.VectorSubcoreMesh(core_axis_name="c", subcore_axis_name="s",
                                  num_cores=NC, num_subcores=NS)
    return pl.kernel(
        _sc_body,
        out_type=jax.ShapeDtypeStruct((NC, NPAD, D), jnp.float32),
        mesh=mesh,
        scratch_types=[
            pltpu.VMEM((EPT,), jnp.int32),
            pltpu.VMEM((K, C), jnp.int32),
            pltpu.VMEM((C, D), jnp.float32),
            pltpu.VMEM((C, D), jnp.float32),
            pltpu.VMEM_SHARED((NPAD, D), jnp.float32),
            pltpu.SemaphoreType.DMA,
            pltpu.SemaphoreType.DMA,
        ],
    )


TF = 1000


def _final_body(root_ref, o_ref, out_ref):
    out_ref[...] = root_ref[...] + o_ref[0] + o_ref[1]


def _final(root_pb, o):
    return pl.pallas_call(
        _final_body,
        grid=(N // TF,),
        in_specs=[
            pl.BlockSpec((TF, D), lambda i: (i, 0)),
            pl.BlockSpec((NC, TF, D), lambda i: (0, i, 0)),
        ],
        out_specs=pl.BlockSpec((TF, D), lambda i: (i, 0)),
        out_shape=jax.ShapeDtypeStruct((N, D), jnp.float32),
    )(root_pb, o)


def kernel(h, edge_index, rel_type, weight, w_comp, root_weight, bias):
    ei = edge_index.astype(jnp.int32)
    rel = rel_type.astype(jnp.int32)

    w3 = _basis(w_comp, weight.reshape(NB, D * D)).reshape(R, D, D)
    t, root_pb = _transform(h, w3, root_weight, bias.reshape(1, D))
    table = t.reshape(R * N, D)

    key2 = (rel * N + ei[0]).reshape(NW, EPT)
    ei4 = ei.reshape(2, NW, K, C)
    zrows = jnp.zeros((RPS, D), jnp.float32)

    o = _sc_edge_agg()(table, key2, ei4, zrows)

    return _final(root_pb, o)


# speedup vs baseline: 1.5028x; 1.5028x over previous
import functools

import jax
import jax.numpy as jnp
from jax import lax
from jax.experimental import pallas as pl
from jax.experimental.pallas import tpu as pltpu
from jax.experimental.pallas import tpu_sc as plsc

N = 10000
E = 320000
D = 128
R = 16
NB = 8

NC = 2
NS = 16
NW = NC * NS
EPT = E // NW
C = 80
K = EPT // C
NPAD = 10240
RPS = NPAD // NS
EL = E // D


TN = 400


def _transform_body(wc_ref, w3_ref, rw_ref, bias_ref, ei_ref, rel_ref, h_ref,
                    t_ref, root_ref, key_ref, wt_ref):
    i = pl.program_id(0)

    @pl.when(i == 0)
    def _():
        for r in range(R):
            acc = wc_ref[r, 0] * w3_ref[0]
            for b in range(1, NB):
                acc = acc + wc_ref[r, b] * w3_ref[b]
            wt_ref[:, r * D:(r + 1) * D] = acc
        key_ref[...] = ei_ref[0] * R + rel_ref[...]

    h = h_ref[...].astype(jnp.bfloat16)
    t_ref[...] = jnp.dot(h, wt_ref[...].astype(jnp.bfloat16),
                         preferred_element_type=jnp.float32)
    root_ref[...] = (jnp.dot(h, rw_ref[...].astype(jnp.bfloat16),
                             preferred_element_type=jnp.float32)
                     + bias_ref[...])


def _transform(w_comp, weight3, root_weight, bias2d, ei3, rel2, h):
    return pl.pallas_call(
        _transform_body,
        grid=(N // TN,),
        in_specs=[
            pl.BlockSpec(memory_space=pltpu.SMEM),
            pl.BlockSpec((NB, D, D), lambda i: (0, 0, 0)),
            pl.BlockSpec((D, D), lambda i: (0, 0)),
            pl.BlockSpec((1, D), lambda i: (0, 0)),
            pl.BlockSpec((2, EL, D), lambda i: (0, 0, 0)),
            pl.BlockSpec((EL, D), lambda i: (0, 0)),
            pl.BlockSpec((TN, D), lambda i: (i, 0)),
        ],
        out_specs=[
            pl.BlockSpec((TN, R * D), lambda i: (i, 0)),
            pl.BlockSpec((TN, D), lambda i: (i, 0)),
            pl.BlockSpec((EL, D), lambda i: (0, 0)),
        ],
        out_shape=[
            jax.ShapeDtypeStruct((N, R * D), jnp.float32),
            jax.ShapeDtypeStruct((N, D), jnp.float32),
            jax.ShapeDtypeStruct((EL, D), jnp.int32),
        ],
        scratch_shapes=[pltpu.VMEM((D, R * D), jnp.float32)],
    )(w_comp, weight3, root_weight, bias2d, ei3, rel2, h)


def _sc_body(table_hbm, key2_hbm, ei4_hbm, zrows_hbm, out_hbm,
             keys1, dst2, rows0, rows1, acc, semg0, semg1):
    c = lax.axis_index("c")
    s = lax.axis_index("s")
    wid = s * NC + c
    pltpu.sync_copy(key2_hbm.at[wid], keys1)
    pltpu.sync_copy(ei4_hbm.at[1, wid], dst2)
    pltpu.sync_copy(zrows_hbm, acc.at[pl.ds(s * RPS, RPS)])
    plsc.subcore_barrier()

    pltpu.async_copy(table_hbm.at[keys1.at[pl.ds(0, C)]], rows0, semg0)

    def pair(p, carry):
        j0 = 2 * p
        pltpu.async_copy(table_hbm.at[keys1.at[pl.ds((j0 + 1) * C, C)]], rows1, semg1)
        pltpu.make_async_copy(table_hbm.at[keys1.at[pl.ds(j0 * C, C)]], rows0, semg0).wait()
        pltpu.sync_copy(rows0, acc.at[dst2.at[j0]], add=True)
        pltpu.async_copy(table_hbm.at[keys1.at[pl.ds((j0 + 2) * C, C)]], rows0, semg0)
        pltpu.make_async_copy(table_hbm.at[keys1.at[pl.ds((j0 + 1) * C, C)]], rows1,
                              semg1).wait()
        pltpu.sync_copy(rows1, acc.at[dst2.at[j0 + 1]], add=True)
        return carry

    lax.fori_loop(0, (K - 1) // 2, pair, 0)
    pltpu.make_async_copy(table_hbm.at[keys1.at[pl.ds((K - 1) * C, C)]], rows0, semg0).wait()
    pltpu.sync_copy(rows0, acc.at[dst2.at[K - 1]], add=True)
    plsc.subcore_barrier()
    pltpu.sync_copy(acc.at[pl.ds(s * RPS, RPS)],
                    out_hbm.at[c, pl.ds(s * RPS, RPS)])


@functools.cache
def _sc_edge_agg():
    mesh = plsc.VectorSubcoreMesh(core_axis_name="c", subcore_axis_name="s",
                                  num_cores=NC, num_subcores=NS)
    return pl.kernel(
        _sc_body,
        out_type=jax.ShapeDtypeStruct((NC, NPAD, D), jnp.float32),
        mesh=mesh,
        scratch_types=[
            pltpu.VMEM((EPT,), jnp.int32),
            pltpu.VMEM((K, C), jnp.int32),
            pltpu.VMEM((C, D), jnp.float32),
            pltpu.VMEM((C, D), jnp.float32),
            pltpu.VMEM_SHARED((NPAD, D), jnp.float32),
            pltpu.SemaphoreType.DMA,
            pltpu.SemaphoreType.DMA,
        ],
    )


TF = 1000


def _final_body(root_ref, o_ref, out_ref):
    out_ref[...] = root_ref[...] + o_ref[0] + o_ref[1]


def _final(root_pb, o):
    return pl.pallas_call(
        _final_body,
        grid=(N // TF,),
        in_specs=[
            pl.BlockSpec((TF, D), lambda i: (i, 0)),
            pl.BlockSpec((NC, TF, D), lambda i: (0, i, 0)),
        ],
        out_specs=pl.BlockSpec((TF, D), lambda i: (i, 0)),
        out_shape=jax.ShapeDtypeStruct((N, D), jnp.float32),
    )(root_pb, o)


def kernel(h, edge_index, rel_type, weight, w_comp, root_weight, bias):
    ei = edge_index.astype(jnp.int32)
    rel = rel_type.astype(jnp.int32)

    t, root_pb, key = _transform(w_comp, weight, root_weight,
                                 bias.reshape(1, D), ei.reshape(2, EL, D),
                                 rel.reshape(EL, D), h)
    table = t.reshape(N * R, D)
    key2 = key.reshape(NW, EPT)
    ei4 = ei.reshape(2, NW, K, C)
    zrows = jnp.zeros((RPS, D), jnp.float32)

    o = _sc_edge_agg()(table, key2, ei4, zrows)

    return _final(root_pb, o)


# speedup vs baseline: 1.5044x; 1.0011x over previous
import functools

import jax
import jax.numpy as jnp
from jax import lax
from jax.experimental import pallas as pl
from jax.experimental.pallas import tpu as pltpu
from jax.experimental.pallas import tpu_sc as plsc

N = 10000
E = 320000
D = 128
R = 16
NB = 8

NC = 2
NS = 16
NW = NC * NS
EPT = E // NW
C = 80
K = EPT // C
NPAD = 10240
RPS = NPAD // NS
EL = E // D


TN = 400


def _transform_body(wc_ref, w3_ref, rw_ref, bias_ref, ei_ref, rel_ref, h_ref,
                    t_ref, root_ref, key_ref, wt_ref):
    i = pl.program_id(0)

    @pl.when(i == 0)
    def _():
        for r in range(R):
            acc = wc_ref[r, 0] * w3_ref[0]
            for b in range(1, NB):
                acc = acc + wc_ref[r, b] * w3_ref[b]
            wt_ref[:, r * D:(r + 1) * D] = acc
        key_ref[...] = ei_ref[0] * R + rel_ref[...]

    h = h_ref[...].astype(jnp.bfloat16)
    t_ref[...] = jnp.dot(h, wt_ref[...].astype(jnp.bfloat16),
                         preferred_element_type=jnp.float32)
    root_ref[...] = (jnp.dot(h, rw_ref[...].astype(jnp.bfloat16),
                             preferred_element_type=jnp.float32)
                     + bias_ref[...])


def _transform(w_comp, weight3, root_weight, bias2d, ei3, rel2, h):
    return pl.pallas_call(
        _transform_body,
        grid=(N // TN,),
        in_specs=[
            pl.BlockSpec(memory_space=pltpu.SMEM),
            pl.BlockSpec((NB, D, D), lambda i: (0, 0, 0)),
            pl.BlockSpec((D, D), lambda i: (0, 0)),
            pl.BlockSpec((1, D), lambda i: (0, 0)),
            pl.BlockSpec((2, EL, D), lambda i: (0, 0, 0)),
            pl.BlockSpec((EL, D), lambda i: (0, 0)),
            pl.BlockSpec((TN, D), lambda i: (i, 0)),
        ],
        out_specs=[
            pl.BlockSpec((TN, R * D), lambda i: (i, 0)),
            pl.BlockSpec((TN, D), lambda i: (i, 0)),
            pl.BlockSpec((EL, D), lambda i: (0, 0)),
        ],
        out_shape=[
            jax.ShapeDtypeStruct((N, R * D), jnp.float32),
            jax.ShapeDtypeStruct((N, D), jnp.float32),
            jax.ShapeDtypeStruct((EL, D), jnp.int32),
        ],
        scratch_shapes=[pltpu.VMEM((D, R * D), jnp.float32)],
    )(w_comp, weight3, root_weight, bias2d, ei3, rel2, h)


def _sc_body(table_hbm, key2_hbm, dst3_hbm, zrows_hbm, out_hbm,
             keys1, dst2, rows0, rows1, acc, semg0, semg1):
    c = lax.axis_index("c")
    s = lax.axis_index("s")
    wid = s * NC + c
    pltpu.sync_copy(key2_hbm.at[wid], keys1)
    pltpu.sync_copy(dst3_hbm.at[1, wid], dst2)
    pltpu.sync_copy(zrows_hbm, rows0)
    for q in range(RPS // C):
        pltpu.sync_copy(rows0, acc.at[pl.ds(s * RPS + q * C, C)])
    plsc.subcore_barrier()

    pltpu.async_copy(table_hbm.at[keys1.at[pl.ds(0, C)]], rows0, semg0)

    def pair(p, carry):
        j0 = 2 * p
        pltpu.async_copy(table_hbm.at[keys1.at[pl.ds((j0 + 1) * C, C)]], rows1, semg1)
        pltpu.make_async_copy(table_hbm.at[keys1.at[pl.ds(j0 * C, C)]], rows0, semg0).wait()
        pltpu.sync_copy(rows0, acc.at[dst2.at[j0]], add=True)
        pltpu.async_copy(table_hbm.at[keys1.at[pl.ds((j0 + 2) * C, C)]], rows0, semg0)
        pltpu.make_async_copy(table_hbm.at[keys1.at[pl.ds((j0 + 1) * C, C)]], rows1,
                              semg1).wait()
        pltpu.sync_copy(rows1, acc.at[dst2.at[j0 + 1]], add=True)
        return carry

    lax.fori_loop(0, (K - 1) // 2, pair, 0)
    pltpu.make_async_copy(table_hbm.at[keys1.at[pl.ds((K - 1) * C, C)]], rows0, semg0).wait()
    pltpu.sync_copy(rows0, acc.at[dst2.at[K - 1]], add=True)
    plsc.subcore_barrier()
    pltpu.sync_copy(acc.at[pl.ds(s * RPS, RPS)],
                    out_hbm.at[c, pl.ds(s * RPS, RPS)])


@functools.cache
def _sc_edge_agg():
    mesh = plsc.VectorSubcoreMesh(core_axis_name="c", subcore_axis_name="s",
                                  num_cores=NC, num_subcores=NS)
    return pl.kernel(
        _sc_body,
        out_type=jax.ShapeDtypeStruct((NC, NPAD, D), jnp.float32),
        mesh=mesh,
        scratch_types=[
            pltpu.VMEM((EPT,), jnp.int32),
            pltpu.VMEM((K, C), jnp.int32),
            pltpu.VMEM((C, D), jnp.float32),
            pltpu.VMEM((C, D), jnp.float32),
            pltpu.VMEM_SHARED((NPAD, D), jnp.float32),
            pltpu.SemaphoreType.DMA,
            pltpu.SemaphoreType.DMA,
        ],
    )


TF = 1000


def _final_body(root_ref, o_ref, out_ref):
    out_ref[...] = root_ref[...] + o_ref[0] + o_ref[1]


def _final(root_pb, o):
    return pl.pallas_call(
        _final_body,
        grid=(N // TF,),
        in_specs=[
            pl.BlockSpec((TF, D), lambda i: (i, 0)),
            pl.BlockSpec((NC, TF, D), lambda i: (0, i, 0)),
        ],
        out_specs=pl.BlockSpec((TF, D), lambda i: (i, 0)),
        out_shape=jax.ShapeDtypeStruct((N, D), jnp.float32),
    )(root_pb, o)


def kernel(h, edge_index, rel_type, weight, w_comp, root_weight, bias):
    ei = edge_index.astype(jnp.int32)
    rel = rel_type.astype(jnp.int32)

    t, root_pb, key = _transform(w_comp, weight, root_weight,
                                 bias.reshape(1, D), ei.reshape(2, EL, D),
                                 rel.reshape(EL, D), h)
    table = t.reshape(N * R, D)
    key2 = key.reshape(NW, EPT)
    dst3 = ei.reshape(2, NW, K, C)
    zrows = jnp.zeros((C, D), jnp.float32)

    o = _sc_edge_agg()(table, key2, dst3, zrows)

    return _final(root_pb, o)
